# baseline (speedup 1.0000x reference)
import functools

import jax
import jax.numpy as jnp
from jax import lax
from jax.experimental import pallas as pl
from jax.experimental.pallas import tpu as pltpu
from jax.experimental.pallas import tpu_sc as plsc

BATCH = 4096
HIST = 200
EMBED_DIM = 64
NTOK = BATCH * HIST


@functools.cache
def _make_gather(V, D):
  info = plsc.get_sparse_core_info()
  nw = info.num_cores * info.num_subcores
  b_per_w = NTOK // nw
  G = 128
  steps = b_per_w // G

  mesh = plsc.VectorSubcoreMesh(core_axis_name="c", subcore_axis_name="s")

  @functools.partial(
      pl.kernel,
      mesh=mesh,
      compiler_params=pltpu.CompilerParams(use_tc_tiling_on_sc=False),
      out_type=jax.ShapeDtypeStruct((NTOK, D), jnp.float32),
      scratch_types=[
          pltpu.VMEM((G,), jnp.int32),
          pltpu.VMEM((G, D), jnp.float32),
          pltpu.SemaphoreType.DMA,
      ],
  )
  def gather_kernel(table_hbm, idx_hbm, out_hbm, idx_v, rows_v, sem):
    wid = lax.axis_index("s") * info.num_cores + lax.axis_index("c")
    base = wid * b_per_w

    def step(g, carry):
      off = base + g * G
      pltpu.sync_copy(idx_hbm.at[pl.ds(off, G)], idx_v)
      pltpu.async_copy(table_hbm.at[idx_v], rows_v, sem).wait()
      pltpu.sync_copy(rows_v, out_hbm.at[pl.ds(off, G)])
      return carry

    lax.fori_loop(0, steps, step, 0)

  return gather_kernel


def kernel(seqTensor, table):
  idx = seqTensor.reshape(-1).astype(jnp.int32)
  out = _make_gather(table.shape[0], EMBED_DIM)(table, idx)
  return out.reshape(BATCH, HIST, EMBED_DIM)


# speedup vs baseline: 1.1749x; 1.1749x over previous
import functools

import jax
import jax.numpy as jnp
from jax import lax
from jax.experimental import pallas as pl
from jax.experimental.pallas import tpu as pltpu
from jax.experimental.pallas import tpu_sc as plsc

BATCH = 4096
HIST = 200
EMBED_DIM = 64
NTOK = BATCH * HIST

G = 128
K = 4


@functools.cache
def _make_gather(V, D):
  info = plsc.get_sparse_core_info()
  nw = info.num_cores * info.num_subcores
  b_per_w = NTOK // nw
  ngroups = b_per_w // (K * G)
  assert ngroups % 2 == 0

  mesh = plsc.VectorSubcoreMesh(core_axis_name="c", subcore_axis_name="s")

  @functools.partial(
      pl.kernel,
      mesh=mesh,
      compiler_params=pltpu.CompilerParams(use_tc_tiling_on_sc=False),
      out_type=jax.ShapeDtypeStruct((NTOK, D), jnp.float32),
      scratch_types=[
          pltpu.VMEM((K * G,), jnp.int32),
          pltpu.VMEM((K * G,), jnp.int32),
          pltpu.VMEM((K, G, D), jnp.float32),
          pltpu.VMEM((K, G, D), jnp.float32),
          pltpu.SemaphoreType.DMA,
          pltpu.SemaphoreType.DMA,
      ],
  )
  def gather_kernel(table_hbm, idx_hbm, out_hbm,
                    idx_a, idx_b, rows_a, rows_b, sem_a, sem_b):
    wid = lax.axis_index("s") * info.num_cores + lax.axis_index("c")
    base = wid * b_per_w

    def fire(t, idx_v, rows_v, sem):
      off = base + t * (K * G)
      pltpu.sync_copy(idx_hbm.at[pl.ds(off, K * G)], idx_v)
      for j in range(K):
        pltpu.async_copy(table_hbm.at[idx_v.at[pl.ds(j * G, G)]],
                         rows_v.at[j], sem)

    def drain_write(t, idx_v, rows_v, sem):
      for j in range(K):
        pltpu.make_async_copy(table_hbm.at[idx_v.at[pl.ds(j * G, G)]],
                              rows_v.at[j], sem).wait()
      off = base + t * (K * G)
      for j in range(K):
        pltpu.sync_copy(rows_v.at[j], out_hbm.at[pl.ds(off + j * G, G)])

    fire(0, idx_a, rows_a, sem_a)

    def body(i, carry):
      t = 2 * i
      fire(t + 1, idx_b, rows_b, sem_b)
      drain_write(t, idx_a, rows_a, sem_a)
      fire(t + 2, idx_a, rows_a, sem_a)
      drain_write(t + 1, idx_b, rows_b, sem_b)
      return carry

    lax.fori_loop(0, (ngroups - 2) // 2, body, 0)

    fire(ngroups - 1, idx_b, rows_b, sem_b)
    drain_write(ngroups - 2, idx_a, rows_a, sem_a)
    drain_write(ngroups - 1, idx_b, rows_b, sem_b)

  return gather_kernel


def kernel(seqTensor, table):
  idx = seqTensor.reshape(-1).astype(jnp.int32)
  out = _make_gather(table.shape[0], EMBED_DIM)(table, idx)
  return out.reshape(BATCH, HIST, EMBED_DIM)
